# Initial kernel scaffold; baseline (speedup 1.0000x reference)
import jax
import jax.numpy as jnp
from jax.experimental import pallas as pl


def kernel(x, xyz, W1, b1, g1, be1, W2, b2, Wm1, bm1, gm, bem, Wm2, bm2):
    raise NotImplementedError("write your pallas kernel here")


# speedup vs baseline: 32.9829x; 32.9829x over previous
import functools

import jax
import jax.numpy as jnp
from jax.experimental import pallas as pl

_K = 16
_CH = 16
_TWO_PI_3 = 2.0943951023931953
_PI = 3.141592653589793


def _acos(r):
    ax = jnp.abs(r)
    p = jnp.float32(-0.0012624911)
    for c in (0.0066700901, -0.0170881256, 0.0308918810, -0.0501743046,
              0.0889789874, -0.2145988016, 1.5707963050):
        p = p * ax + jnp.float32(c)
    acos_pos = jnp.sqrt(jnp.maximum(1.0 - ax, 0.0)) * p
    return jnp.where(r >= 0, acos_pos, _PI - acos_pos)


def _knn_feat_kernel(xyz_blk_ref, xyzt_ref, xyz_all_ref, w1t_ref, b1_ref,
                     h1_ref, stats_ref, *, n_total, r_blk):
    k = _K
    xyz_b = xyz_blk_ref[0]
    xyzt = xyzt_ref[0]
    xyz_all = xyz_all_ref[0]

    xb0 = xyz_b[:, 0:1]
    xb1 = xyz_b[:, 1:2]
    xb2 = xyz_b[:, 2:3]
    xt0 = xyzt[0:1, :]
    xt1 = xyzt[1:2, :]
    xt2 = xyzt[2:3, :]
    sq_b = xb0 * xb0 + xb1 * xb1 + xb2 * xb2
    sq_all = xt0 * xt0 + xt1 * xt1 + xt2 * xt2
    inner = jnp.dot(xyz_b, xyzt, preferred_element_type=jnp.float32)
    d2 = (sq_b + sq_all) - 2.0 * inner

    iota = jax.lax.broadcasted_iota(jnp.int32, (r_blk, n_total), 1)

    rels = []
    for _ in range(k):
        m = jnp.min(d2, axis=1, keepdims=True)
        ism = d2 <= m
        idx = jnp.min(jnp.where(ism, iota, n_total), axis=1, keepdims=True)
        sel = iota == idx
        self_f = sel.astype(jnp.float32)
        nb = jnp.dot(self_f, xyz_all, preferred_element_type=jnp.float32)
        rels.append(nb - xyz_b)
        d2 = jnp.where(sel, jnp.float32(jnp.inf), d2)

    xs = [r[:, 0:1] for r in rels]
    ys = [r[:, 1:2] for r in rels]
    zs = [r[:, 2:3] for r in rels]

    sx = sum(xs); sy = sum(ys); sz = sum(zs)
    cx = sx / k; cy = sy / k; cz = sz / k
    cxx = sum(v * v for v in xs)
    cyy = sum(v * v for v in ys)
    czz = sum(v * v for v in zs)
    cxy = sum(a * b for a, b in zip(xs, ys))
    cxz = sum(a * b for a, b in zip(xs, zs))
    cyz = sum(a * b for a, b in zip(ys, zs))
    km1 = jnp.float32(k - 1)
    a11 = cxx / km1; a22 = cyy / km1; a33 = czz / km1
    a12 = cxy / km1; a13 = cxz / km1; a23 = cyz / km1

    q = (a11 + a22 + a33) / 3.0
    p1 = a12 * a12 + a13 * a13 + a23 * a23
    p2 = (a11 - q) ** 2 + (a22 - q) ** 2 + (a33 - q) ** 2 + 2.0 * p1
    p = jnp.sqrt(p2 / 6.0) + jnp.float32(1e-30)
    b11 = (a11 - q) / p; b22 = (a22 - q) / p; b33 = (a33 - q) / p
    b12 = a12 / p; b13 = a13 / p; b23 = a23 / p
    detb = (b11 * (b22 * b33 - b23 * b23)
            - b12 * (b12 * b33 - b23 * b13)
            + b13 * (b12 * b23 - b22 * b13))
    r = jnp.clip(detb * 0.5, -1.0, 1.0)
    phi = _acos(r) / 3.0
    e2 = q + 2.0 * p * jnp.cos(phi)
    e0 = q + 2.0 * p * jnp.cos(phi + _TWO_PI_3)
    e1 = 3.0 * q - e2 - e0

    c2 = a11 + a22 + a33
    c1 = (a11 * a22 - a12 * a12) + (a11 * a33 - a13 * a13) + (a22 * a33 - a23 * a23)
    c0 = (a11 * (a22 * a33 - a23 * a23)
          - a12 * (a12 * a33 - a23 * a13)
          + a13 * (a12 * a23 - a22 * a13))

    def _polish(lam):
        f = ((-lam + c2) * lam - c1) * lam + c0
        fp = (-3.0 * lam + 2.0 * c2) * lam - c1
        return lam - jnp.where(jnp.abs(fp) > 1e-12, f / fp, jnp.float32(0.0))

    e0 = _polish(e0); e1 = _polish(e1); e2 = _polish(e2)

    eps = jnp.float32(1e-8)
    inv0 = 1.0 / (e0 + eps)
    lin = (e0 - e1) * inv0
    pla = (e1 - e2) * inv0
    sph = e2 * inv0

    dists = []
    for j in range(k):
        dxc = xs[j] - cx; dyc = ys[j] - cy; dzc = zs[j] - cz
        dists.append(jnp.sqrt(dxc * dxc + dyc * dyc + dzc * dzc))
    dmax = functools.reduce(jnp.maximum, dists)
    dmean = sum(dists) / k
    dvar = sum((dj - dmean) ** 2 for dj in dists) / km1
    dstd = jnp.sqrt(dvar)

    nsx = jnp.zeros_like(cx); nsy = jnp.zeros_like(cx); nsz = jnp.zeros_like(cx)
    for j in range(k):
        nrm = jnp.sqrt(xs[j] * xs[j] + ys[j] * ys[j] + zs[j] * zs[j])
        inv = 1.0 / (nrm + eps)
        nsx = nsx + xs[j] * inv
        nsy = nsy + ys[j] * inv
        nsz = nsz + zs[j] * inv
    dcons = (nsx * nsx + nsy * nsy + nsz * nsz) / jnp.float32(k * k)

    zvar = sum((zj - cz) ** 2 for zj in zs) / km1
    zstd = jnp.sqrt(zvar)
    zmax = functools.reduce(jnp.maximum, zs)
    zmin = functools.reduce(jnp.minimum, zs)
    zrange = zmax - zmin

    vx = sum((v - cx) ** 2 for v in xs) / km1
    vy = sum((v - cy) ** 2 for v in ys) / km1
    vz = sum((v - cz) ** 2 for v in zs) / km1
    spread = jnp.sqrt(vx + vy + vz)

    feat = jnp.concatenate(
        [lin, pla, sph, dmax, dmean, dstd, dcons, zstd, zrange,
         cx, cy, cz, spread], axis=1)

    g = jnp.floor(xyz_b)
    encs = []
    for f in (1.0, 2.0, 4.0, 8.0):
        encs.append(jnp.sin(g * f))
        encs.append(jnp.cos(g * f))
    abs_enc = jnp.concatenate(encs, axis=1)

    w1t = w1t_ref[...]
    b1 = b1_ref[...]
    base = (jnp.dot(abs_enc, w1t[0:24, :], preferred_element_type=jnp.float32)
            + jnp.dot(feat, w1t[27:40, :], preferred_element_type=jnp.float32)
            + b1)
    w1b = w1t[24:27, :]

    hs = []
    ssum = jnp.zeros((1, _CH), jnp.float32)
    ssq = jnp.zeros((1, _CH), jnp.float32)
    for j in range(k):
        hj = base + jnp.dot(rels[j], w1b, preferred_element_type=jnp.float32)
        hs.append(hj)
        ssum = ssum + jnp.sum(hj, axis=0, keepdims=True)
        ssq = ssq + jnp.sum(hj * hj, axis=0, keepdims=True)

    h1_ref[0] = jnp.concatenate(hs, axis=1)

    first = (pl.program_id(0) == 0) & (pl.program_id(1) == 0)

    @pl.when(first)
    def _():
        stats_ref[...] = jnp.zeros_like(stats_ref)

    stats_ref[...] += jnp.concatenate([ssum, ssq], axis=0)


def _mid_kernel(h1_ref, xt_ref, stats1_ref, g1_ref, be1_ref, w2t_ref, b2_ref,
                wm1xt_ref, wm1pt_ref, bm1_ref, hm_ref, stats2_ref, *, cnt1):
    k = _K
    s = stats1_ref[...]
    mu = s[0:1, :] / cnt1
    var = s[1:2, :] / cnt1 - mu * mu
    a1 = g1_ref[...] * jax.lax.rsqrt(var + 1e-5)
    c1 = be1_ref[...] - mu * a1

    h1 = h1_ref[0]
    w2t = w2t_ref[...]
    b2 = b2_ref[...]
    pos = None
    for j in range(k):
        hj = h1[:, j * _CH:(j + 1) * _CH] * a1 + c1
        hj = jnp.maximum(hj, 0.0)
        oj = jnp.dot(hj, w2t, preferred_element_type=jnp.float32) + b2
        pos = oj if pos is None else jnp.maximum(pos, oj)

    hm = (jnp.dot(xt_ref[0], wm1xt_ref[...], preferred_element_type=jnp.float32)
          + jnp.dot(pos, wm1pt_ref[...], preferred_element_type=jnp.float32)
          + bm1_ref[...])
    hm_ref[0] = hm

    ssum = jnp.sum(hm, axis=0, keepdims=True)
    ssq = jnp.sum(hm * hm, axis=0, keepdims=True)

    first = (pl.program_id(0) == 0) & (pl.program_id(1) == 0)

    @pl.when(first)
    def _():
        stats2_ref[...] = jnp.zeros_like(stats2_ref)

    stats2_ref[...] += jnp.concatenate([ssum, ssq], axis=0)


def _out_kernel(hm_ref, stats2_ref, gm_ref, bem_ref, wm2t_ref, bm2_ref,
                out_ref, *, cnt2):
    s = stats2_ref[...]
    mu = s[0:1, :] / cnt2
    var = s[1:2, :] / cnt2 - mu * mu
    a2 = gm_ref[...] * jax.lax.rsqrt(var + 1e-5)
    c2 = bem_ref[...] - mu * a2
    h = jnp.maximum(hm_ref[0] * a2 + c2, 0.0)
    out_ref[0] = (jnp.dot(h, wm2t_ref[...], preferred_element_type=jnp.float32)
                  + bm2_ref[...])


def kernel(x, xyz, W1, b1, g1, be1, W2, b2, Wm1, bm1, gm, bem, Wm2, bm2):
    B, C, N = x.shape
    K, CH = _K, _CH
    f32 = jnp.float32

    xyzt = jnp.transpose(xyz, (0, 2, 1))
    xT = jnp.transpose(x, (0, 2, 1))
    w1t = W1.T
    w2t = W2.T
    wm1xt = Wm1[:, :C].T
    wm1pt = Wm1[:, C:].T
    wm2t = Wm2.T
    b1r = b1.reshape(1, CH)
    g1r = g1.reshape(1, CH)
    be1r = be1.reshape(1, CH)
    b2r = b2.reshape(1, CH)
    bm1r = bm1.reshape(1, C)
    gmr = gm.reshape(1, C)
    bemr = bem.reshape(1, C)
    bm2r = bm2.reshape(1, C)

    RA = 256
    h1, stats1 = pl.pallas_call(
        functools.partial(_knn_feat_kernel, n_total=N, r_blk=RA),
        grid=(B, N // RA),
        in_specs=[
            pl.BlockSpec((1, RA, 3), lambda b, n: (b, n, 0)),
            pl.BlockSpec((1, 3, N), lambda b, n: (b, 0, 0)),
            pl.BlockSpec((1, N, 3), lambda b, n: (b, 0, 0)),
            pl.BlockSpec((40, CH), lambda b, n: (0, 0)),
            pl.BlockSpec((1, CH), lambda b, n: (0, 0)),
        ],
        out_specs=[
            pl.BlockSpec((1, RA, K * CH), lambda b, n: (b, n, 0)),
            pl.BlockSpec((2, CH), lambda b, n: (0, 0)),
        ],
        out_shape=[
            jax.ShapeDtypeStruct((B, N, K * CH), f32),
            jax.ShapeDtypeStruct((2, CH), f32),
        ],
    )(xyz, xyzt, xyz, w1t, b1r)

    RB = min(1024, N)
    hm, stats2 = pl.pallas_call(
        functools.partial(_mid_kernel, cnt1=float(B * N * K)),
        grid=(B, N // RB),
        in_specs=[
            pl.BlockSpec((1, RB, K * CH), lambda b, n: (b, n, 0)),
            pl.BlockSpec((1, RB, C), lambda b, n: (b, n, 0)),
            pl.BlockSpec((2, CH), lambda b, n: (0, 0)),
            pl.BlockSpec((1, CH), lambda b, n: (0, 0)),
            pl.BlockSpec((1, CH), lambda b, n: (0, 0)),
            pl.BlockSpec((CH, CH), lambda b, n: (0, 0)),
            pl.BlockSpec((1, CH), lambda b, n: (0, 0)),
            pl.BlockSpec((C, C), lambda b, n: (0, 0)),
            pl.BlockSpec((CH, C), lambda b, n: (0, 0)),
            pl.BlockSpec((1, C), lambda b, n: (0, 0)),
        ],
        out_specs=[
            pl.BlockSpec((1, RB, C), lambda b, n: (b, n, 0)),
            pl.BlockSpec((2, C), lambda b, n: (0, 0)),
        ],
        out_shape=[
            jax.ShapeDtypeStruct((B, N, C), f32),
            jax.ShapeDtypeStruct((2, C), f32),
        ],
    )(h1, xT, stats1, g1r, be1r, w2t, b2r, wm1xt, wm1pt, bm1r)

    RC = min(2048, N)
    outT = pl.pallas_call(
        functools.partial(_out_kernel, cnt2=float(B * N)),
        grid=(B, N // RC),
        in_specs=[
            pl.BlockSpec((1, RC, C), lambda b, n: (b, n, 0)),
            pl.BlockSpec((2, C), lambda b, n: (0, 0)),
            pl.BlockSpec((1, C), lambda b, n: (0, 0)),
            pl.BlockSpec((1, C), lambda b, n: (0, 0)),
            pl.BlockSpec((C, C), lambda b, n: (0, 0)),
            pl.BlockSpec((1, C), lambda b, n: (0, 0)),
        ],
        out_specs=pl.BlockSpec((1, RC, C), lambda b, n: (b, n, 0)),
        out_shape=jax.ShapeDtypeStruct((B, N, C), f32),
    )(hm, stats2, gmr, bemr, wm2t, bm2r)

    return jnp.transpose(outT, (0, 2, 1))
